# baseline (speedup 1.0000x reference)
import functools

import jax
import jax.numpy as jnp
from jax import lax
from jax.experimental import pallas as pl
from jax.experimental.pallas import tpu as pltpu
from jax.experimental.pallas import tpu_sc as plsc

_E = 8
_D = 768
_T = 2048
_BLK = 512
_NB = (_T + _E * _BLK) // _BLK
_PAD = _NB * _BLK
_NC = 2
_NS = 16
_NW = _NC * _NS
_CH = _T // _NW
_GW = 128


def _router_z_body(x_ref, wg_ref, z_ref, g16_ref, pos_ref, bexp_ref, xc_ref):
    x = x_ref[...] + z_ref[0, 0]
    xc_ref[...] = x
    logits = jnp.dot(x, wg_ref[...], preferred_element_type=jnp.float32)
    _router_common(logits, g16_ref, pos_ref, bexp_ref)


def _router_l_body(lg_ref, g16_ref, pos_ref, bexp_ref):
    _router_common(lg_ref[:, :_E], g16_ref, pos_ref, bexp_ref)


def _router_common(logits, g16_ref, pos_ref, bexp_ref):
    m = jnp.max(logits, axis=-1, keepdims=True)
    ssum = jnp.sum(jnp.exp(logits - m), axis=-1, keepdims=True)
    g = 1.0 / ssum
    col = lax.broadcasted_iota(jnp.int32, (_T, _E), 1)
    eidx = jnp.min(jnp.where(logits >= m, col, _E), axis=-1, keepdims=True)
    oh = (col == eidx).astype(jnp.int32)

    c = oh
    sh = 1
    while sh < _T:
        c = c + jnp.concatenate(
            [jnp.zeros((sh, _E), jnp.int32), c[: _T - sh, :]], axis=0)
        sh *= 2
    counts = c[_T - 1 : _T, :]
    padded = ((counts + (_BLK - 1)) // _BLK) * _BLK

    ao = jnp.concatenate(
        [jnp.zeros((1, 1), jnp.int32), padded[:, : _E - 1]], axis=1)
    shl = 1
    while shl < _E:
        ao = ao + jnp.concatenate(
            [jnp.zeros((1, shl), jnp.int32), ao[:, : _E - shl]], axis=1)
        shl *= 2
    ao_end = ao + padded

    pos = (jnp.sum(oh * ao, axis=-1, keepdims=True)
           + jnp.sum(oh * c, axis=-1, keepdims=True) - 1)
    pos_ref[...] = jnp.reshape(pos, (_T // 128, 128))
    g16_ref[...] = jnp.broadcast_to(g, (_T, _GW))

    brow = lax.broadcasted_iota(jnp.int32, (128, _E), 0) * _BLK
    be = jnp.sum((brow >= ao_end).astype(jnp.int32), axis=-1, keepdims=True)
    be = jnp.minimum(be, _E - 1)
    rid = lax.broadcasted_iota(jnp.int32, (128, 1), 0)
    used_blocks = ao_end[:, _E - 1 :] // _BLK
    valid = rid < used_blocks
    be_prev = jnp.concatenate([be[:1], be[:-1]], axis=0)
    chg = (valid & (rid > 0) & (be != be_prev)).astype(jnp.int32)
    isstart = ((rid == 0) | (chg == 1)).astype(jnp.int32)
    slot = chg
    shc = 1
    while shc < 128:
        slot = slot + jnp.concatenate(
            [jnp.zeros((shc, 1), jnp.int32), slot[: 128 - shc]], axis=0)
        shc *= 2
    slot = slot % 2
    present = counts > 0
    col8 = lax.broadcasted_iota(jnp.int32, (1, _E), 1)
    v = jnp.where(present, col8, _E)

    def _shl(a, k):
        return jnp.concatenate(
            [a[:, k:], jnp.full((1, k), _E, jnp.int32)], axis=1)

    r = _shl(v, 1)
    r = jnp.minimum(r, _shl(r, 1))
    r = jnp.minimum(r, _shl(r, 2))
    r = jnp.minimum(r, _shl(r, 4))
    nexte_e = jnp.where(r > _E - 1, -1, r)
    ohb = (lax.broadcasted_iota(jnp.int32, (128, _E), 1) == be).astype(jnp.int32)
    nexte = jnp.sum(ohb * nexte_e, axis=-1, keepdims=True)
    c0 = jnp.where(rid == _NB, used_blocks, be)
    bexp_ref[...] = jnp.transpose(
        jnp.concatenate([c0, slot, isstart, nexte], axis=1))


_router_l = pl.pallas_call(
    _router_l_body,
    out_shape=[
        jax.ShapeDtypeStruct((_T, _GW), jnp.float32),
        jax.ShapeDtypeStruct((_T // 128, 128), jnp.int32),
        jax.ShapeDtypeStruct((4, 128), jnp.int32),
    ],
)

_router_z = pl.pallas_call(
    _router_z_body,
    out_shape=[
        jax.ShapeDtypeStruct((_T, _GW), jnp.float32),
        jax.ShapeDtypeStruct((_T // 128, 128), jnp.int32),
        jax.ShapeDtypeStruct((4, 128), jnp.int32),
        jax.ShapeDtypeStruct((_T, _D), jnp.float32),
    ],
)


def _wid():
    return lax.axis_index("s") * _NC + lax.axis_index("c")


@functools.lru_cache(maxsize=None)
def _sc_kernels():
    mesh = plsc.VectorSubcoreMesh(
        core_axis_name="c", subcore_axis_name="s", num_cores=_NC)

    @functools.partial(
        pl.kernel,
        mesh=mesh,
        out_type=[
            jax.ShapeDtypeStruct((_PAD, _D), jnp.float32),
            jax.ShapeDtypeStruct((_PAD, _GW), jnp.float32),
        ],
        scratch_types=[
            pltpu.VMEM((_CH,), jnp.int32),
            pltpu.VMEM((_CH, _D), jnp.float32),
            pltpu.VMEM((_CH, _GW), jnp.float32),
            pltpu.SemaphoreType.DMA,
        ],
    )
    def sc_scatter(x_hbm, pos_hbm, g16_hbm, xb_hbm, gb_hbm,
                   idx_v, rows_v, g_v, sem):
        base = _wid() * _CH
        ld0 = pltpu.async_copy(pos_hbm.at[pl.ds(base, _CH)], idx_v, sem)
        ld1 = pltpu.async_copy(x_hbm.at[pl.ds(base, _CH)], rows_v, sem)
        ld2 = pltpu.async_copy(g16_hbm.at[pl.ds(base, _CH)], g_v, sem)
        ld0.wait(); ld1.wait(); ld2.wait()
        st0 = pltpu.async_copy(rows_v, xb_hbm.at[idx_v], sem)
        st1 = pltpu.async_copy(g_v, gb_hbm.at[idx_v], sem)
        st0.wait(); st1.wait()

    @functools.partial(
        pl.kernel,
        mesh=mesh,
        out_type=jax.ShapeDtypeStruct((_T, _GW), jnp.float32),
        scratch_types=[
            pltpu.VMEM((_CH,), jnp.int32),
            pltpu.VMEM((_CH, _GW), jnp.float32),
            pltpu.SemaphoreType.DMA,
        ],
    )
    def sc_gather_l(src_hbm, pos_hbm, out_hbm, idx_v, rows_v, sem):
        base = _wid() * _CH
        pltpu.sync_copy(pos_hbm.at[pl.ds(base, _CH)], idx_v)
        pltpu.async_copy(src_hbm.at[idx_v], rows_v, sem).wait()
        pltpu.sync_copy(rows_v, out_hbm.at[pl.ds(base, _CH)])

    @functools.partial(
        pl.kernel,
        mesh=mesh,
        out_type=[
            jax.ShapeDtypeStruct((_PAD, _D), jnp.float32),
            jax.ShapeDtypeStruct((_PAD, _GW), jnp.float32),
        ],
        scratch_types=[
            pltpu.VMEM((_CH,), jnp.int32),
            pltpu.VMEM((_CH,), jnp.int32),
            pltpu.VMEM((_CH, _D), jnp.float32),
            pltpu.VMEM((_CH, _GW), jnp.float32),
            pltpu.SemaphoreType.DMA,
        ],
    )
    def sc_gather_scatter(src_hbm, pos0_hbm, pos1_hbm, g16_hbm, xb_hbm, gb_hbm,
                          idx0_v, idx1_v, rows_v, g_v, sem):
        base = _wid() * _CH
        ld0 = pltpu.async_copy(pos0_hbm.at[pl.ds(base, _CH)], idx0_v, sem)
        ld1 = pltpu.async_copy(pos1_hbm.at[pl.ds(base, _CH)], idx1_v, sem)
        ld2 = pltpu.async_copy(g16_hbm.at[pl.ds(base, _CH)], g_v, sem)
        ld0.wait(); ld1.wait(); ld2.wait()
        gat = pltpu.async_copy(src_hbm.at[idx0_v], rows_v, sem)
        st1 = pltpu.async_copy(g_v, gb_hbm.at[idx1_v], sem)
        gat.wait()
        st0 = pltpu.async_copy(rows_v, xb_hbm.at[idx1_v], sem)
        st1.wait(); st0.wait()

    @functools.partial(
        pl.kernel,
        mesh=mesh,
        out_type=[jax.ShapeDtypeStruct((_T, _D), jnp.float32)] * 3,
        scratch_types=[
            pltpu.VMEM((_CH,), jnp.int32),
            pltpu.VMEM((_CH, _D), jnp.float32),
            pltpu.VMEM((_CH, _D), jnp.float32),
            pltpu.SemaphoreType.DMA,
        ],
    )
    def sc_gather3(a_hbm, b_hbm, c_hbm, pos_hbm, oa_hbm, ob_hbm, oc_hbm,
                   idx_v, rows0_v, rows1_v, sem):
        base = _wid() * _CH
        pltpu.sync_copy(pos_hbm.at[pl.ds(base, _CH)], idx_v)
        bufs = (rows0_v, rows1_v)
        srcs = (a_hbm, b_hbm, c_hbm)
        dsts = (oa_hbm, ob_hbm, oc_hbm)
        pend = pltpu.async_copy(srcs[0].at[idx_v], bufs[0], sem)
        for i in range(3):
            pend.wait()
            if i < 2:
                pend = pltpu.async_copy(srcs[i + 1].at[idx_v], bufs[(i + 1) % 2], sem)
            pltpu.sync_copy(bufs[i % 2], dsts[i].at[pl.ds(base, _CH)])

    return sc_scatter, sc_gather_l, sc_gather_scatter, sc_gather3


def _make_mlp_body(nw):
    def body(bctl_ref, xb_ref, gb_ref, *rest):
        whbms = rest[:nw]
        if nw == 3:
            wg_ref = rest[nw]
            outs = rest[nw + 1:nw + 3]
        else:
            outs = rest[nw:nw + 3]
        wbuf, sem0, sem1 = rest[-3], rest[-2], rest[-1]
        b = pl.program_id(0)
        e = bctl_ref[b]
        sl = bctl_ref[128 + b]
        strt = bctl_ref[256 + b]
        nxt = bctl_ref[384 + b]
        used = bctl_ref[_NB]

        def fire(expert, slot, sem):
            for i, wh in enumerate(whbms):
                pltpu.make_async_copy(wh.at[expert], wbuf.at[slot, i], sem).start()

        def drain1(slot, i, sem):
            pltpu.make_async_copy(whbms[i].at[0], wbuf.at[slot, i], sem).wait()

        @pl.when(b == 0)
        def _():
            fire(e, 0, sem0)

        def drain_sl(i):
            @pl.when((strt == 1) & (sl == 0))
            def _():
                drain1(0, i, sem0)

            @pl.when((strt == 1) & (sl == 1))
            def _():
                drain1(1, i, sem1)

        @pl.when(b < used)
        def _():
            @pl.when((strt == 1) & (nxt >= 0) & (sl == 0))
            def _():
                fire(nxt, 1, sem1)

            @pl.when((strt == 1) & (nxt >= 0) & (sl == 1))
            def _():
                fire(nxt, 0, sem0)

            g = gb_ref[:, 0:1]
            drain_sl(0)
            h = jnp.maximum(
                jnp.dot(xb_ref[...], wbuf[sl, 0],
                        preferred_element_type=jnp.float32), 0.0)
            drain_sl(1)
            h = jnp.dot(h, wbuf[sl, 1], preferred_element_type=jnp.float32)
            drain_sl(2)
            mu = jnp.dot(h, wbuf[sl, 2], preferred_element_type=jnp.float32)
            mug = mu * g
            outs[0][...] = mug
            if nw == 3:
                lg = jnp.dot(mug, wg_ref[...],
                             preferred_element_type=jnp.float32)
                outs[1][...] = jnp.concatenate(
                    [lg, jnp.zeros((_BLK, _GW - _E), jnp.float32)], axis=1)
            else:
                drain_sl(3)
                outs[1][...] = jnp.dot(
                    h, wbuf[sl, 3], preferred_element_type=jnp.float32) * g
                drain_sl(4)
                outs[2][...] = jnp.dot(
                    mu, wbuf[sl, 4], preferred_element_type=jnp.float32) * g
    return body


def _clampb(b, s):
    return (jnp.minimum(b, s[_NB] - 1), 0)


_xspec = pl.BlockSpec((_BLK, _D), _clampb)
_gspec = pl.BlockSpec((_BLK, _GW), _clampb)
_wspec = pl.BlockSpec(memory_space=pl.ANY)
_ospec = pl.BlockSpec((_BLK, _D), _clampb)

_wgspec = pl.BlockSpec((_D, _E), lambda b, s: (0, 0))
_lspec = pl.BlockSpec((_BLK, _GW), _clampb)

_mlp_first = pl.pallas_call(
    _make_mlp_body(3),
    grid_spec=pltpu.PrefetchScalarGridSpec(
        num_scalar_prefetch=1,
        grid=(_NB,),
        in_specs=[_xspec, _gspec, _wspec, _wspec, _wspec, _wgspec],
        out_specs=[_ospec, _lspec],
        scratch_shapes=[
            pltpu.VMEM((2, 3, _D, _D), jnp.float32),
            pltpu.SemaphoreType.DMA,
            pltpu.SemaphoreType.DMA,
        ],
    ),
    out_shape=[
        jax.ShapeDtypeStruct((_PAD, _D), jnp.float32),
        jax.ShapeDtypeStruct((_PAD, _GW), jnp.float32),
    ],
)

_mlp_last = pl.pallas_call(
    _make_mlp_body(5),
    grid_spec=pltpu.PrefetchScalarGridSpec(
        num_scalar_prefetch=1,
        grid=(_NB,),
        in_specs=[_xspec, _gspec, _wspec, _wspec, _wspec, _wspec, _wspec],
        out_specs=[_ospec, _ospec, _ospec],
        scratch_shapes=[
            pltpu.VMEM((2, 5, _D, _D), jnp.float32),
            pltpu.SemaphoreType.DMA,
            pltpu.SemaphoreType.DMA,
        ],
    ),
    out_shape=[jax.ShapeDtypeStruct((_PAD, _D), jnp.float32)] * 3,
)


def kernel(x, params, deep):
    sc_scatter, sc_gather_l, sc_gather_scatter, sc_gather3 = _sc_kernels()
    zarr = jnp.asarray(deep - 2, jnp.float32).reshape(1, 1)

    p0, p1 = params[0], params[1]
    g16, pos0, bctl, xcur = _router_z(x, p0["Wg"], zarr)
    pos0, bctl = pos0.reshape(_T), bctl.reshape(512)
    xb, gb = sc_scatter(xcur, pos0, g16)
    mub0, lgb = _mlp_first(bctl, xb, gb, p0["Wmlp"][0], p0["Wmlp"][1],
                           p0["Wmu"], p1["Wg"])
    lg1 = sc_gather_l(lgb, pos0)

    g16, pos1, bctl = _router_l(lg1)
    pos1, bctl = pos1.reshape(_T), bctl.reshape(512)
    xb, gb = sc_gather_scatter(mub0, pos0, pos1, g16)
    mub, lvb, xhb = _mlp_last(bctl, xb, gb, p1["Wmlp"][0], p1["Wmlp"][1],
                              p1["Wmu"], p1["Wlv"], p1["Wrec"])
    mu, lv, xh = sc_gather3(mub, lvb, xhb, pos1)
    return (mu, lv, xh)


# speedup vs baseline: 1.0048x; 1.0048x over previous
import functools

import jax
import jax.numpy as jnp
from jax import lax
from jax.experimental import pallas as pl
from jax.experimental.pallas import tpu as pltpu
from jax.experimental.pallas import tpu_sc as plsc

_E = 8
_D = 768
_T = 2048
_BLK = 256
_NB = (_T + _E * _BLK) // _BLK
_PAD = _NB * _BLK
_NC = 2
_NS = 16
_NW = _NC * _NS
_CH = _T // _NW
_GW = 128


def _router_z_body(x_ref, wg_ref, z_ref, g16_ref, pos_ref, bexp_ref, xc_ref):
    x = x_ref[...] + z_ref[0, 0]
    xc_ref[...] = x
    logits = jnp.dot(x, wg_ref[...], preferred_element_type=jnp.float32)
    _router_common(logits, g16_ref, pos_ref, bexp_ref)


def _router_l_body(lg_ref, g16_ref, pos_ref, bexp_ref):
    _router_common(lg_ref[:, :_E], g16_ref, pos_ref, bexp_ref)


def _router_common(logits, g16_ref, pos_ref, bexp_ref):
    m = jnp.max(logits, axis=-1, keepdims=True)
    ssum = jnp.sum(jnp.exp(logits - m), axis=-1, keepdims=True)
    g = 1.0 / ssum
    col = lax.broadcasted_iota(jnp.int32, (_T, _E), 1)
    eidx = jnp.min(jnp.where(logits >= m, col, _E), axis=-1, keepdims=True)
    oh = (col == eidx).astype(jnp.int32)

    c = oh
    sh = 1
    while sh < _T:
        c = c + jnp.concatenate(
            [jnp.zeros((sh, _E), jnp.int32), c[: _T - sh, :]], axis=0)
        sh *= 2
    counts = c[_T - 1 : _T, :]
    padded = ((counts + (_BLK - 1)) // _BLK) * _BLK

    ao = jnp.concatenate(
        [jnp.zeros((1, 1), jnp.int32), padded[:, : _E - 1]], axis=1)
    shl = 1
    while shl < _E:
        ao = ao + jnp.concatenate(
            [jnp.zeros((1, shl), jnp.int32), ao[:, : _E - shl]], axis=1)
        shl *= 2
    ao_end = ao + padded

    pos = (jnp.sum(oh * ao, axis=-1, keepdims=True)
           + jnp.sum(oh * c, axis=-1, keepdims=True) - 1)
    pos_ref[...] = jnp.reshape(pos, (_T // 128, 128))
    g16_ref[...] = jnp.broadcast_to(g, (_T, _GW))

    brow = lax.broadcasted_iota(jnp.int32, (128, _E), 0) * _BLK
    be = jnp.sum((brow >= ao_end).astype(jnp.int32), axis=-1, keepdims=True)
    be = jnp.minimum(be, _E - 1)
    rid = lax.broadcasted_iota(jnp.int32, (128, 1), 0)
    used_blocks = ao_end[:, _E - 1 :] // _BLK
    valid = rid < used_blocks
    be_prev = jnp.concatenate([be[:1], be[:-1]], axis=0)
    chg = (valid & (rid > 0) & (be != be_prev)).astype(jnp.int32)
    isstart = ((rid == 0) | (chg == 1)).astype(jnp.int32)
    slot = chg
    shc = 1
    while shc < 128:
        slot = slot + jnp.concatenate(
            [jnp.zeros((shc, 1), jnp.int32), slot[: 128 - shc]], axis=0)
        shc *= 2
    slot = slot % 2
    present = counts > 0
    col8 = lax.broadcasted_iota(jnp.int32, (1, _E), 1)
    v = jnp.where(present, col8, _E)

    def _shl(a, k):
        return jnp.concatenate(
            [a[:, k:], jnp.full((1, k), _E, jnp.int32)], axis=1)

    r = _shl(v, 1)
    r = jnp.minimum(r, _shl(r, 1))
    r = jnp.minimum(r, _shl(r, 2))
    r = jnp.minimum(r, _shl(r, 4))
    nexte_e = jnp.where(r > _E - 1, -1, r)
    ohb = (lax.broadcasted_iota(jnp.int32, (128, _E), 1) == be).astype(jnp.int32)
    nexte = jnp.sum(ohb * nexte_e, axis=-1, keepdims=True)
    c0 = jnp.where(rid == _NB, used_blocks, be)
    bexp_ref[...] = jnp.transpose(
        jnp.concatenate([c0, slot, isstart, nexte], axis=1))


_router_l = pl.pallas_call(
    _router_l_body,
    out_shape=[
        jax.ShapeDtypeStruct((_T, _GW), jnp.float32),
        jax.ShapeDtypeStruct((_T // 128, 128), jnp.int32),
        jax.ShapeDtypeStruct((4, 128), jnp.int32),
    ],
)

_router_z = pl.pallas_call(
    _router_z_body,
    out_shape=[
        jax.ShapeDtypeStruct((_T, _GW), jnp.float32),
        jax.ShapeDtypeStruct((_T // 128, 128), jnp.int32),
        jax.ShapeDtypeStruct((4, 128), jnp.int32),
        jax.ShapeDtypeStruct((_T, _D), jnp.float32),
    ],
)


def _wid():
    return lax.axis_index("s") * _NC + lax.axis_index("c")


@functools.lru_cache(maxsize=None)
def _sc_kernels():
    mesh = plsc.VectorSubcoreMesh(
        core_axis_name="c", subcore_axis_name="s", num_cores=_NC)

    @functools.partial(
        pl.kernel,
        mesh=mesh,
        out_type=[
            jax.ShapeDtypeStruct((_PAD, _D), jnp.float32),
            jax.ShapeDtypeStruct((_PAD, _GW), jnp.float32),
        ],
        scratch_types=[
            pltpu.VMEM((_CH,), jnp.int32),
            pltpu.VMEM((_CH, _D), jnp.float32),
            pltpu.VMEM((_CH, _GW), jnp.float32),
            pltpu.SemaphoreType.DMA,
        ],
    )
    def sc_scatter(x_hbm, pos_hbm, g16_hbm, xb_hbm, gb_hbm,
                   idx_v, rows_v, g_v, sem):
        base = _wid() * _CH
        ld0 = pltpu.async_copy(pos_hbm.at[pl.ds(base, _CH)], idx_v, sem)
        ld1 = pltpu.async_copy(x_hbm.at[pl.ds(base, _CH)], rows_v, sem)
        ld2 = pltpu.async_copy(g16_hbm.at[pl.ds(base, _CH)], g_v, sem)
        ld0.wait(); ld1.wait(); ld2.wait()
        st0 = pltpu.async_copy(rows_v, xb_hbm.at[idx_v], sem)
        st1 = pltpu.async_copy(g_v, gb_hbm.at[idx_v], sem)
        st0.wait(); st1.wait()

    @functools.partial(
        pl.kernel,
        mesh=mesh,
        out_type=jax.ShapeDtypeStruct((_T, _GW), jnp.float32),
        scratch_types=[
            pltpu.VMEM((_CH,), jnp.int32),
            pltpu.VMEM((_CH, _GW), jnp.float32),
            pltpu.SemaphoreType.DMA,
        ],
    )
    def sc_gather_l(src_hbm, pos_hbm, out_hbm, idx_v, rows_v, sem):
        base = _wid() * _CH
        pltpu.sync_copy(pos_hbm.at[pl.ds(base, _CH)], idx_v)
        pltpu.async_copy(src_hbm.at[idx_v], rows_v, sem).wait()
        pltpu.sync_copy(rows_v, out_hbm.at[pl.ds(base, _CH)])

    @functools.partial(
        pl.kernel,
        mesh=mesh,
        out_type=[
            jax.ShapeDtypeStruct((_PAD, _D), jnp.float32),
            jax.ShapeDtypeStruct((_PAD, _GW), jnp.float32),
        ],
        scratch_types=[
            pltpu.VMEM((_CH,), jnp.int32),
            pltpu.VMEM((_CH,), jnp.int32),
            pltpu.VMEM((_CH, _D), jnp.float32),
            pltpu.VMEM((_CH, _GW), jnp.float32),
            pltpu.SemaphoreType.DMA,
        ],
    )
    def sc_gather_scatter(src_hbm, pos0_hbm, pos1_hbm, g16_hbm, xb_hbm, gb_hbm,
                          idx0_v, idx1_v, rows_v, g_v, sem):
        base = _wid() * _CH
        ld0 = pltpu.async_copy(pos0_hbm.at[pl.ds(base, _CH)], idx0_v, sem)
        ld1 = pltpu.async_copy(pos1_hbm.at[pl.ds(base, _CH)], idx1_v, sem)
        ld2 = pltpu.async_copy(g16_hbm.at[pl.ds(base, _CH)], g_v, sem)
        ld0.wait(); ld1.wait(); ld2.wait()
        gat = pltpu.async_copy(src_hbm.at[idx0_v], rows_v, sem)
        st1 = pltpu.async_copy(g_v, gb_hbm.at[idx1_v], sem)
        gat.wait()
        st0 = pltpu.async_copy(rows_v, xb_hbm.at[idx1_v], sem)
        st1.wait(); st0.wait()

    @functools.partial(
        pl.kernel,
        mesh=mesh,
        out_type=[jax.ShapeDtypeStruct((_T, _D), jnp.float32)] * 3,
        scratch_types=[
            pltpu.VMEM((_CH,), jnp.int32),
            pltpu.VMEM((_CH, _D), jnp.float32),
            pltpu.VMEM((_CH, _D), jnp.float32),
            pltpu.SemaphoreType.DMA,
        ],
    )
    def sc_gather3(a_hbm, b_hbm, c_hbm, pos_hbm, oa_hbm, ob_hbm, oc_hbm,
                   idx_v, rows0_v, rows1_v, sem):
        base = _wid() * _CH
        pltpu.sync_copy(pos_hbm.at[pl.ds(base, _CH)], idx_v)
        bufs = (rows0_v, rows1_v)
        srcs = (a_hbm, b_hbm, c_hbm)
        dsts = (oa_hbm, ob_hbm, oc_hbm)
        pend = pltpu.async_copy(srcs[0].at[idx_v], bufs[0], sem)
        for i in range(3):
            pend.wait()
            if i < 2:
                pend = pltpu.async_copy(srcs[i + 1].at[idx_v], bufs[(i + 1) % 2], sem)
            pltpu.sync_copy(bufs[i % 2], dsts[i].at[pl.ds(base, _CH)])

    return sc_scatter, sc_gather_l, sc_gather_scatter, sc_gather3


def _make_mlp_body(nw):
    def body(bctl_ref, xb_ref, gb_ref, *rest):
        whbms = rest[:nw]
        if nw == 3:
            wg_ref = rest[nw]
            outs = rest[nw + 1:nw + 3]
        else:
            outs = rest[nw:nw + 3]
        wbuf, sem0, sem1 = rest[-3], rest[-2], rest[-1]
        b = pl.program_id(0)
        e = bctl_ref[b]
        sl = bctl_ref[128 + b]
        strt = bctl_ref[256 + b]
        nxt = bctl_ref[384 + b]
        used = bctl_ref[_NB]

        def fire(expert, slot, sem):
            for i, wh in enumerate(whbms):
                pltpu.make_async_copy(wh.at[expert], wbuf.at[slot, i], sem).start()

        def drain1(slot, i, sem):
            pltpu.make_async_copy(whbms[i].at[0], wbuf.at[slot, i], sem).wait()

        @pl.when(b == 0)
        def _():
            fire(e, 0, sem0)

        def drain_sl(i):
            @pl.when((strt == 1) & (sl == 0))
            def _():
                drain1(0, i, sem0)

            @pl.when((strt == 1) & (sl == 1))
            def _():
                drain1(1, i, sem1)

        @pl.when(b < used)
        def _():
            @pl.when((strt == 1) & (nxt >= 0) & (sl == 0))
            def _():
                fire(nxt, 1, sem1)

            @pl.when((strt == 1) & (nxt >= 0) & (sl == 1))
            def _():
                fire(nxt, 0, sem0)

            g = gb_ref[:, 0:1]
            drain_sl(0)
            h = jnp.maximum(
                jnp.dot(xb_ref[...], wbuf[sl, 0],
                        preferred_element_type=jnp.float32), 0.0)
            drain_sl(1)
            h = jnp.dot(h, wbuf[sl, 1], preferred_element_type=jnp.float32)
            drain_sl(2)
            mu = jnp.dot(h, wbuf[sl, 2], preferred_element_type=jnp.float32)
            mug = mu * g
            outs[0][...] = mug
            if nw == 3:
                lg = jnp.dot(mug, wg_ref[...],
                             preferred_element_type=jnp.float32)
                outs[1][...] = jnp.concatenate(
                    [lg, jnp.zeros((_BLK, _GW - _E), jnp.float32)], axis=1)
            else:
                drain_sl(3)
                outs[1][...] = jnp.dot(
                    h, wbuf[sl, 3], preferred_element_type=jnp.float32) * g
                drain_sl(4)
                outs[2][...] = jnp.dot(
                    mu, wbuf[sl, 4], preferred_element_type=jnp.float32) * g
    return body


def _clampb(b, s):
    return (jnp.minimum(b, s[_NB] - 1), 0)


_xspec = pl.BlockSpec((_BLK, _D), _clampb)
_gspec = pl.BlockSpec((_BLK, _GW), _clampb)
_wspec = pl.BlockSpec(memory_space=pl.ANY)
_ospec = pl.BlockSpec((_BLK, _D), _clampb)

_wgspec = pl.BlockSpec((_D, _E), lambda b, s: (0, 0))
_lspec = pl.BlockSpec((_BLK, _GW), _clampb)

_mlp_first = pl.pallas_call(
    _make_mlp_body(3),
    grid_spec=pltpu.PrefetchScalarGridSpec(
        num_scalar_prefetch=1,
        grid=(_NB,),
        in_specs=[_xspec, _gspec, _wspec, _wspec, _wspec, _wgspec],
        out_specs=[_ospec, _lspec],
        scratch_shapes=[
            pltpu.VMEM((2, 3, _D, _D), jnp.float32),
            pltpu.SemaphoreType.DMA,
            pltpu.SemaphoreType.DMA,
        ],
    ),
    out_shape=[
        jax.ShapeDtypeStruct((_PAD, _D), jnp.float32),
        jax.ShapeDtypeStruct((_PAD, _GW), jnp.float32),
    ],
)

_mlp_last = pl.pallas_call(
    _make_mlp_body(5),
    grid_spec=pltpu.PrefetchScalarGridSpec(
        num_scalar_prefetch=1,
        grid=(_NB,),
        in_specs=[_xspec, _gspec, _wspec, _wspec, _wspec, _wspec, _wspec],
        out_specs=[_ospec, _ospec, _ospec],
        scratch_shapes=[
            pltpu.VMEM((2, 5, _D, _D), jnp.float32),
            pltpu.SemaphoreType.DMA,
            pltpu.SemaphoreType.DMA,
        ],
    ),
    out_shape=[jax.ShapeDtypeStruct((_PAD, _D), jnp.float32)] * 3,
)


def kernel(x, params, deep):
    sc_scatter, sc_gather_l, sc_gather_scatter, sc_gather3 = _sc_kernels()
    zarr = jnp.asarray(deep - 2, jnp.float32).reshape(1, 1)

    p0, p1 = params[0], params[1]
    g16, pos0, bctl, xcur = _router_z(x, p0["Wg"], zarr)
    pos0, bctl = pos0.reshape(_T), bctl.reshape(512)
    xb, gb = sc_scatter(xcur, pos0, g16)
    mub0, lgb = _mlp_first(bctl, xb, gb, p0["Wmlp"][0], p0["Wmlp"][1],
                           p0["Wmu"], p1["Wg"])
    lg1 = sc_gather_l(lgb, pos0)

    g16, pos1, bctl = _router_l(lg1)
    pos1, bctl = pos1.reshape(_T), bctl.reshape(512)
    xb, gb = sc_gather_scatter(mub0, pos0, pos1, g16)
    mub, lvb, xhb = _mlp_last(bctl, xb, gb, p1["Wmlp"][0], p1["Wmlp"][1],
                              p1["Wmu"], p1["Wlv"], p1["Wrec"])
    mu, lv, xh = sc_gather3(mub, lvb, xhb, pos1)
    return (mu, lv, xh)
